# baseline (device time: 35281 ns/iter reference)
import jax
import jax.numpy as jnp
from jax import lax
from jax.experimental import pallas as pl
from jax.experimental.pallas import tpu as pltpu

SIZES = [32, 64, 96, 128, 128, 128, 128, 128, 96, 48, 32, 16]
C = len(SIZES)
OFFS = [sum(SIZES[:i]) for i in range(C)]


def kernel(x):
    _, m, n = x.shape
    assert sum(SIZES) == m

    def body(x_ref, out_ref, recv_x, sx_sems, rx_sems, sy_sems, ry_sems):
        my_x = lax.axis_index("x")
        my_y = lax.axis_index("y")
        x_partner = (1 - my_x, my_y)
        y_partner = (my_x, 1 - my_y)
        my_cols = pl.ds(my_y * n, n)

        barrier_sem = pltpu.get_barrier_semaphore()
        pl.semaphore_signal(
            barrier_sem, inc=1, device_id=x_partner,
            device_id_type=pl.DeviceIdType.MESH,
        )
        pl.semaphore_signal(
            barrier_sem, inc=1, device_id=y_partner,
            device_id_type=pl.DeviceIdType.MESH,
        )
        pl.semaphore_wait(barrier_sem, 2)

        p1 = []
        for i in range(C):
            sl = pl.ds(OFFS[i], SIZES[i])
            r = pltpu.make_async_remote_copy(
                src_ref=x_ref.at[0, sl, :],
                dst_ref=recv_x.at[sl, :],
                send_sem=sx_sems.at[i],
                recv_sem=rx_sems.at[i],
                device_id=x_partner,
                device_id_type=pl.DeviceIdType.MESH,
            )
            r.start()
            p1.append(r)

        p2 = []
        for i in range(C):
            sl = pl.ds(OFFS[i], SIZES[i])
            p1[i].wait_recv()
            out_ref[sl, my_cols] = x_ref[0, sl, :] + recv_x[sl, :]
            r2 = pltpu.make_async_remote_copy(
                src_ref=out_ref.at[sl, my_cols],
                dst_ref=out_ref.at[sl, my_cols],
                send_sem=sy_sems.at[i],
                recv_sem=ry_sems.at[i],
                device_id=y_partner,
                device_id_type=pl.DeviceIdType.MESH,
            )
            r2.start()
            p2.append(r2)

        for i in range(C):
            p2[i].wait_recv()
        for i in range(C):
            p1[i].wait_send()
            p2[i].wait_send()

    return pl.pallas_call(
        body,
        out_shape=jax.ShapeDtypeStruct((m, 2 * n), jnp.float32),
        in_specs=[pl.BlockSpec(memory_space=pltpu.VMEM)],
        out_specs=pl.BlockSpec(memory_space=pltpu.VMEM),
        scratch_shapes=[
            pltpu.VMEM((m, n), jnp.float32),
            pltpu.SemaphoreType.DMA((C,)),
            pltpu.SemaphoreType.DMA((C,)),
            pltpu.SemaphoreType.DMA((C,)),
            pltpu.SemaphoreType.DMA((C,)),
        ],
        compiler_params=pltpu.CompilerParams(collective_id=0),
    )(x)


# device time: 34255 ns/iter; 1.0300x vs baseline; 1.0300x over previous
import jax
import jax.numpy as jnp
from jax import lax
from jax.experimental import pallas as pl
from jax.experimental.pallas import tpu as pltpu

C = 16


def kernel(x):
    _, m, n = x.shape
    rows = m // C

    def body(x_ref, out_ref, recv_x, part,
             sx_sems, rx_sems, sy_sems, ry_sems, loc_sems):
        my_x = lax.axis_index("x")
        my_y = lax.axis_index("y")
        x_partner = (1 - my_x, my_y)
        y_partner = (my_x, 1 - my_y)
        my_cols = pl.ds(my_y * n, n)

        barrier_sem = pltpu.get_barrier_semaphore()
        pl.semaphore_signal(
            barrier_sem, inc=1, device_id=x_partner,
            device_id_type=pl.DeviceIdType.MESH,
        )
        pl.semaphore_signal(
            barrier_sem, inc=1, device_id=y_partner,
            device_id_type=pl.DeviceIdType.MESH,
        )
        pl.semaphore_wait(barrier_sem, 2)

        p1 = []
        for i in range(C):
            sl = pl.ds(i * rows, rows)
            r = pltpu.make_async_remote_copy(
                src_ref=x_ref.at[0, sl, :],
                dst_ref=recv_x.at[sl, :],
                send_sem=sx_sems.at[i],
                recv_sem=rx_sems.at[i],
                device_id=x_partner,
                device_id_type=pl.DeviceIdType.MESH,
            )
            r.start()
            p1.append(r)

        p2 = []
        loc = []
        for i in range(C):
            sl = pl.ds(i * rows, rows)
            p1[i].wait_recv()
            part[sl, :] = x_ref[0, sl, :] + recv_x[sl, :]
            c = pltpu.make_async_copy(
                part.at[sl, :], out_ref.at[sl, my_cols], loc_sems.at[i],
            )
            c.start()
            loc.append(c)
            r2 = pltpu.make_async_remote_copy(
                src_ref=part.at[sl, :],
                dst_ref=out_ref.at[sl, my_cols],
                send_sem=sy_sems.at[i],
                recv_sem=ry_sems.at[i],
                device_id=y_partner,
                device_id_type=pl.DeviceIdType.MESH,
            )
            r2.start()
            p2.append(r2)

        for i in range(C):
            p2[i].wait_recv()
        for i in range(C):
            loc[i].wait()
            p1[i].wait_send()
            p2[i].wait_send()

    return pl.pallas_call(
        body,
        out_shape=jax.ShapeDtypeStruct((m, 2 * n), jnp.float32),
        in_specs=[pl.BlockSpec(memory_space=pltpu.VMEM)],
        out_specs=pl.BlockSpec(memory_space=pltpu.HBM),
        scratch_shapes=[
            pltpu.VMEM((m, n), jnp.float32),
            pltpu.VMEM((m, n), jnp.float32),
            pltpu.SemaphoreType.DMA((C,)),
            pltpu.SemaphoreType.DMA((C,)),
            pltpu.SemaphoreType.DMA((C,)),
            pltpu.SemaphoreType.DMA((C,)),
            pltpu.SemaphoreType.DMA((C,)),
        ],
        compiler_params=pltpu.CompilerParams(collective_id=0),
    )(x)


# device time: 34004 ns/iter; 1.0376x vs baseline; 1.0074x over previous
import jax
import jax.numpy as jnp
from jax import lax
from jax.experimental import pallas as pl
from jax.experimental.pallas import tpu as pltpu

C = 16


def kernel(x):
    _, m, n = x.shape
    rows = m // C

    def body(x_ref, out_ref, recv_x, sx_sems, rx_sems, sy_sems, ry_sems):
        my_x = lax.axis_index("x")
        my_y = lax.axis_index("y")
        x_partner = (1 - my_x, my_y)
        y_partner = (my_x, 1 - my_y)
        my_cols = pl.ds(my_y * n, n)

        barrier_sem = pltpu.get_barrier_semaphore()
        pl.semaphore_signal(
            barrier_sem, inc=1, device_id=x_partner,
            device_id_type=pl.DeviceIdType.MESH,
        )
        pl.semaphore_signal(
            barrier_sem, inc=1, device_id=y_partner,
            device_id_type=pl.DeviceIdType.MESH,
        )
        pl.semaphore_wait(barrier_sem, 2)

        p1 = []
        for i in range(C):
            sl = pl.ds(i * rows, rows)
            r = pltpu.make_async_remote_copy(
                src_ref=x_ref.at[0, sl, :],
                dst_ref=recv_x.at[sl, :],
                send_sem=sx_sems.at[i],
                recv_sem=rx_sems.at[i],
                device_id=x_partner,
                device_id_type=pl.DeviceIdType.MESH,
            )
            r.start()
            p1.append(r)

        p2 = []
        for i in range(C):
            sl = pl.ds(i * rows, rows)
            p1[i].wait_recv()
            out_ref[sl, my_cols] = x_ref[0, sl, :] + recv_x[sl, :]
            r2 = pltpu.make_async_remote_copy(
                src_ref=out_ref.at[sl, my_cols],
                dst_ref=out_ref.at[sl, my_cols],
                send_sem=sy_sems.at[i],
                recv_sem=ry_sems.at[i],
                device_id=y_partner,
                device_id_type=pl.DeviceIdType.MESH,
            )
            r2.start()
            p2.append(r2)

        for i in range(C):
            p2[i].wait_recv()
        for i in range(C):
            p1[i].wait_send()
            p2[i].wait_send()

    return pl.pallas_call(
        body,
        out_shape=jax.ShapeDtypeStruct((m, 2 * n), jnp.float32),
        in_specs=[pl.BlockSpec(memory_space=pltpu.VMEM)],
        out_specs=pl.BlockSpec(memory_space=pltpu.VMEM),
        scratch_shapes=[
            pltpu.VMEM((m, n), jnp.float32),
            pltpu.SemaphoreType.DMA((C,)),
            pltpu.SemaphoreType.DMA((C,)),
            pltpu.SemaphoreType.DMA((C,)),
            pltpu.SemaphoreType.DMA((C,)),
        ],
        compiler_params=pltpu.CompilerParams(collective_id=0),
    )(x)
